# Initial kernel scaffold; baseline (speedup 1.0000x reference)
import jax
import jax.numpy as jnp
from jax.experimental import pallas as pl


def kernel(x, W_g, experts):
    raise NotImplementedError("write your pallas kernel here")


# speedup vs baseline: 1.2563x; 1.2563x over previous
import functools

import jax
import jax.numpy as jnp
from jax import lax
from jax.experimental import pallas as pl
from jax.experimental.pallas import tpu as pltpu
from jax.experimental.pallas import tpu_sc as plsc

D = 1024
E = 8
B = 2
N = 2048
M = N // E
P = E * B
R = P * M
SINKHORN_ITERS = 8
EPS = 1e-6


def _lse_lanes(t):
    mx = jnp.max(t, axis=-1, keepdims=True)
    return jnp.log(jnp.sum(jnp.exp(t - mx), axis=-1, keepdims=True)) + mx


def _route_kernel(x_ref, wg_ref, fidx_ref, gate_ref, *,
                  logits_precision=None):
    logits = lax.dot_general(
        wg_ref[...], x_ref[...], (((1,), (1,)), ((), ())),
        precision=logits_precision,
        preferred_element_type=jnp.float32)
    t = jnp.log(jnp.maximum(logits, EPS))
    for _ in range(SINKHORN_ITERS):
        t0 = t[:, :N]
        t1 = t[:, N:]
        t = jnp.concatenate([t0 - _lse_lanes(t0), t1 - _lse_lanes(t1)], axis=1)
        mx = jnp.max(t, axis=0, keepdims=True)
        t = t - (jnp.log(jnp.sum(jnp.exp(t - mx), axis=0, keepdims=True)) + mx)
    g8 = jnp.exp(t)
    g = jnp.concatenate([g8[:, :N], g8[:, N:]], axis=0)

    u = lax.bitcast_convert_type(g, jnp.int32)
    lo = jnp.zeros((P, 1), jnp.int32)
    hi = jnp.full((P, 1), 0x7F800000, jnp.int32)
    for _ in range(31):
        mid = lo + (hi - lo) // 2
        cnt = jnp.sum((u >= mid).astype(jnp.int32), axis=1, keepdims=True)
        ok = cnt >= M
        lo = jnp.where(ok, mid, lo)
        hi = jnp.where(ok, hi, mid)
    thr = lo
    c_gt = jnp.sum((u > thr).astype(jnp.int32), axis=1, keepdims=True)
    need = M - c_gt
    ties = u == thr
    pos = lax.broadcasted_iota(jnp.int32, (P, N), 1)
    lo2 = jnp.zeros((P, 1), jnp.int32)
    hi2 = jnp.full((P, 1), N, jnp.int32)
    for _ in range(12):
        mid = lo2 + (hi2 - lo2) // 2
        cnt = jnp.sum((ties & (pos < mid)).astype(jnp.int32),
                      axis=1, keepdims=True)
        ok = cnt >= need
        hi2 = jnp.where(ok, mid, hi2)
        lo2 = jnp.where(ok, lo2, mid)
    sel = (u > thr) | (ties & (pos < hi2))

    selF = sel.astype(jnp.float32)
    BL = 256
    tri = (lax.broadcasted_iota(jnp.int32, (BL, BL), 0)
           < lax.broadcasted_iota(jnp.int32, (BL, BL), 1)).astype(jnp.float32)
    blocks = []
    carry = jnp.zeros((P, 1), jnp.float32)
    for blk in range(N // BL):
        sb = selF[:, blk * BL:(blk + 1) * BL]
        rb = lax.dot_general(sb, tri, (((1,), (0,)), ((), ())),
                             precision=lax.Precision.HIGHEST,
                             preferred_element_type=jnp.float32) + carry
        blocks.append(rb)
        carry = carry + jnp.sum(sb, axis=1, keepdims=True)
    rank = jnp.concatenate(blocks, axis=1).astype(jnp.int32)

    slot = lax.broadcasted_iota(jnp.int32, (M, N), 0)
    tok = lax.broadcasted_iota(jnp.int32, (1, N), 1)
    for q in range(P):
        b, e = q // E, q % E
        p = e * B + b
        onehot = ((rank[q:q + 1, :] == slot) & sel[q:q + 1, :]).astype(
            jnp.float32)
        pay = jnp.concatenate(
            [(tok + b * N).astype(jnp.float32), g[q:q + 1, :]], axis=0)
        res = lax.dot_general(onehot, pay, (((1,), (1,)), ((), ())),
                              precision=lax.Precision.HIGHEST,
                              preferred_element_type=jnp.float32)
        fidx_ref[:, p:p + 1] = res[:, 0:1].astype(jnp.int32)
        gate_ref[:, p:p + 1] = res[:, 1:2]


_NC = 2
_NS = 16
_NW = _NC * _NS
_ROWS_W = R // _NW
_CH = 32
_NCH = _ROWS_W // _CH


@functools.cache
def _make_gather_kernel():
    @functools.partial(
        pl.kernel,
        mesh=plsc.VectorSubcoreMesh(core_axis_name="c", subcore_axis_name="s"),
        out_type=jax.ShapeDtypeStruct((R, D), jnp.float32),
        scratch_types=[
            pltpu.VMEM((_ROWS_W,), jnp.int32),
            pltpu.VMEM((_CH, D), jnp.float32),
            pltpu.VMEM((_CH, D), jnp.float32),
            pltpu.SemaphoreType.DMA,
            pltpu.SemaphoreType.DMA,
        ],
    )
    def _gather_kernel(x_hbm, fidx_hbm, out_hbm, idx_v, buf0, buf1,
                       sem0, sem1):
        wid = lax.axis_index("s") * _NC + lax.axis_index("c")
        base = wid * _ROWS_W
        pltpu.sync_copy(fidx_hbm.at[pl.ds(base, _ROWS_W)], idx_v)
        bufs = (buf0, buf1)
        sems = (sem0, sem1)
        copies = [None, None]
        copies[0] = pltpu.async_copy(
            x_hbm.at[idx_v.at[pl.ds(0, _CH)]], bufs[0], sems[0])
        for c in range(_NCH):
            cur = c % 2
            nxt = (c + 1) % 2
            if c + 1 < _NCH:
                copies[nxt] = pltpu.async_copy(
                    x_hbm.at[idx_v.at[pl.ds((c + 1) * _CH, _CH)]],
                    bufs[nxt], sems[nxt])
            copies[cur].wait()
            pltpu.sync_copy(bufs[cur], out_hbm.at[pl.ds(base + c * _CH, _CH)])

    return _gather_kernel


def _expert_kernel(fidx_ref, routed_ref, w_ref, gate_ref, out_ref, y_ref):
    e = pl.program_id(0)

    @pl.when(e == 0)
    def _():
        out_ref[...] = jnp.zeros((B * N, D), jnp.float32)

    y = lax.dot_general(routed_ref[...], w_ref[0], (((1,), (0,)), ((), ())),
                        preferred_element_type=jnp.float32)
    y_ref[...] = y * gate_ref[...]

    def body(i, _):
        r = fidx_ref[B * M * e + i]
        out_ref[pl.ds(r, 1), :] = y_ref[pl.ds(i, 1), :]
        return 0

    lax.fori_loop(0, B * M, body, 0)


def kernel(x, W_g, experts):
    x2d = x.reshape(B * N, D)
    fidx_t, gate_t = pl.pallas_call(
        _route_kernel,
        out_shape=[
            jax.ShapeDtypeStruct((M, P), jnp.int32),
            jax.ShapeDtypeStruct((M, P), jnp.float32),
        ],
    )(x2d, W_g)
    fidx = fidx_t.T.reshape(R)
    gates = gate_t.T.reshape(R, 1)
    routed = _make_gather_kernel()(x2d, fidx)
    out = pl.pallas_call(
        _expert_kernel,
        grid=(E,),
        in_specs=[
            pl.BlockSpec(memory_space=pltpu.SMEM),
            pl.BlockSpec((B * M, D), lambda e: (e, 0)),
            pl.BlockSpec((1, D, D), lambda e: (e, 0, 0)),
            pl.BlockSpec((B * M, 1), lambda e: (e, 0)),
        ],
        out_specs=pl.BlockSpec((B * N, D), lambda e: (0, 0)),
        out_shape=jax.ShapeDtypeStruct((B * N, D), jnp.float32),
        scratch_shapes=[pltpu.VMEM((B * M, D), jnp.float32)],
        compiler_params=pltpu.CompilerParams(
            dimension_semantics=("arbitrary",)),
    )(fidx, routed, experts, gates)
    return out.reshape(B, N, D)


# speedup vs baseline: 1.3006x; 1.0352x over previous
import functools

import jax
import jax.numpy as jnp
from jax import lax
from jax.experimental import pallas as pl
from jax.experimental.pallas import tpu as pltpu
from jax.experimental.pallas import tpu_sc as plsc

D = 1024
E = 8
B = 2
N = 2048
M = N // E
P = E * B
R = P * M
SINKHORN_ITERS = 8
EPS = 1e-6
XB = 4


def _lse_lanes(t):
    mx = jnp.max(t, axis=-1, keepdims=True)
    return jnp.log(jnp.sum(jnp.exp(t - mx), axis=-1, keepdims=True)) + mx


def _route_kernel(x_ref, wg_ref, fidx_ref, gate_ref, lscr):
    i = pl.program_id(0)
    lscr[i] = lax.dot_general(
        wg_ref[...], x_ref[...], (((1,), (1,)), ((), ())),
        preferred_element_type=jnp.float32)

    @pl.when(i == XB - 1)
    def _():
        logits = jnp.concatenate([lscr[k] for k in range(XB)], axis=1)
        t = jnp.log(jnp.maximum(logits, EPS))
        for _ in range(SINKHORN_ITERS):
            t0 = t[:, :N]
            t1 = t[:, N:]
            t = jnp.concatenate(
                [t0 - _lse_lanes(t0), t1 - _lse_lanes(t1)], axis=1)
            mx = jnp.max(t, axis=0, keepdims=True)
            t = t - (jnp.log(jnp.sum(jnp.exp(t - mx), axis=0, keepdims=True))
                     + mx)
        g8 = jnp.exp(t)
        g = jnp.concatenate([g8[:, :N], g8[:, N:]], axis=0)

        u = lax.bitcast_convert_type(g, jnp.int32)
        lo = jnp.zeros((P, 1), jnp.int32)
        hi = jnp.full((P, 1), 0x7F800000, jnp.int32)
        for _ in range(31):
            mid = lo + (hi - lo) // 2
            cnt = jnp.sum((u >= mid).astype(jnp.int32), axis=1, keepdims=True)
            ok = cnt >= M
            lo = jnp.where(ok, mid, lo)
            hi = jnp.where(ok, hi, mid)
        thr = lo
        c_gt = jnp.sum((u > thr).astype(jnp.int32), axis=1, keepdims=True)
        need = M - c_gt
        ties = u == thr
        pos = lax.broadcasted_iota(jnp.int32, (P, N), 1)
        lo2 = jnp.zeros((P, 1), jnp.int32)
        hi2 = jnp.full((P, 1), N, jnp.int32)
        for _ in range(12):
            mid = lo2 + (hi2 - lo2) // 2
            cnt = jnp.sum((ties & (pos < mid)).astype(jnp.int32),
                          axis=1, keepdims=True)
            ok = cnt >= need
            hi2 = jnp.where(ok, mid, hi2)
            lo2 = jnp.where(ok, lo2, mid)
        sel = (u > thr) | (ties & (pos < hi2))

        selF = sel.astype(jnp.float32)
        BL = 256
        tri = (lax.broadcasted_iota(jnp.int32, (BL, BL), 0)
               < lax.broadcasted_iota(jnp.int32, (BL, BL), 1)
               ).astype(jnp.float32)
        blocks = []
        carry = jnp.zeros((P, 1), jnp.float32)
        for blk in range(N // BL):
            sb = selF[:, blk * BL:(blk + 1) * BL]
            rb = lax.dot_general(sb, tri, (((1,), (0,)), ((), ())),
                                 precision=lax.Precision.HIGHEST,
                                 preferred_element_type=jnp.float32) + carry
            blocks.append(rb)
            carry = carry + jnp.sum(sb, axis=1, keepdims=True)
        rank = jnp.concatenate(blocks, axis=1).astype(jnp.int32)
        rank = jnp.where(sel, rank, M)

        slot = lax.broadcasted_iota(jnp.int32, (M, N), 0)
        tok = lax.broadcasted_iota(jnp.int32, (1, N), 1)
        for q in range(P):
            b, e = q // E, q % E
            p = e * B + b
            onehot = jnp.where(rank[q:q + 1, :] == slot, 1.0, 0.0)
            pay = jnp.concatenate(
                [(tok + b * N).astype(jnp.float32), g[q:q + 1, :]], axis=0)
            res = lax.dot_general(onehot, pay, (((1,), (1,)), ((), ())),
                                  precision=lax.Precision.HIGHEST,
                                  preferred_element_type=jnp.float32)
            fidx_ref[:, p:p + 1] = res[:, 0:1].astype(jnp.int32)
            gate_ref[:, p:p + 1] = res[:, 1:2]


_NC = 2
_NS = 16
_NW = _NC * _NS
_ROWS_W = R // _NW
_CH = 32
_NCH = _ROWS_W // _CH


@functools.cache
def _make_gather_kernel():
    @functools.partial(
        pl.kernel,
        mesh=plsc.VectorSubcoreMesh(core_axis_name="c", subcore_axis_name="s"),
        out_type=jax.ShapeDtypeStruct((R, D), jnp.float32),
        scratch_types=[
            pltpu.VMEM((_ROWS_W,), jnp.int32),
            pltpu.VMEM((_CH, D), jnp.float32),
            pltpu.VMEM((_CH, D), jnp.float32),
            pltpu.SemaphoreType.DMA,
            pltpu.SemaphoreType.DMA,
        ],
    )
    def _gather_kernel(x_hbm, fidx_hbm, out_hbm, idx_v, buf0, buf1,
                       sem0, sem1):
        wid = lax.axis_index("s") * _NC + lax.axis_index("c")
        base = wid * _ROWS_W
        pltpu.sync_copy(fidx_hbm.at[pl.ds(base, _ROWS_W)], idx_v)
        bufs = (buf0, buf1)
        sems = (sem0, sem1)
        copies = [None, None]
        copies[0] = pltpu.async_copy(
            x_hbm.at[idx_v.at[pl.ds(0, _CH)]], bufs[0], sems[0])
        for c in range(_NCH):
            cur = c % 2
            nxt = (c + 1) % 2
            if c + 1 < _NCH:
                copies[nxt] = pltpu.async_copy(
                    x_hbm.at[idx_v.at[pl.ds((c + 1) * _CH, _CH)]],
                    bufs[nxt], sems[nxt])
            copies[cur].wait()
            pltpu.sync_copy(bufs[cur], out_hbm.at[pl.ds(base + c * _CH, _CH)])

    return _gather_kernel


def _expert_kernel(fidx_ref, routed_ref, w_ref, gate_ref, out_ref, y_ref):
    e = pl.program_id(0)

    @pl.when(e == 0)
    def _():
        out_ref[...] = jnp.zeros((B * N, D), jnp.float32)

    y = lax.dot_general(routed_ref[...], w_ref[0], (((1,), (0,)), ((), ())),
                        preferred_element_type=jnp.float32)
    y_ref[...] = y * gate_ref[...]

    def body(i, _):
        r = fidx_ref[B * M * e + i]
        out_ref[pl.ds(r, 1), :] = y_ref[pl.ds(i, 1), :]
        return 0

    lax.fori_loop(0, B * M, body, 0)


def kernel(x, W_g, experts):
    x2d = x.reshape(B * N, D)
    fidx_t, gate_t = pl.pallas_call(
        _route_kernel,
        grid=(XB,),
        in_specs=[
            pl.BlockSpec((B * N // XB, D), lambda i: (i, 0)),
            pl.BlockSpec((E, D), lambda i: (0, 0)),
        ],
        out_specs=[
            pl.BlockSpec((M, P), lambda i: (0, 0)),
            pl.BlockSpec((M, P), lambda i: (0, 0)),
        ],
        out_shape=[
            jax.ShapeDtypeStruct((M, P), jnp.int32),
            jax.ShapeDtypeStruct((M, P), jnp.float32),
        ],
        scratch_shapes=[pltpu.VMEM((XB, E, B * N // XB), jnp.float32)],
        compiler_params=pltpu.CompilerParams(
            dimension_semantics=("arbitrary",)),
    )(x2d, W_g)
    fidx = fidx_t.T.reshape(R)
    gates = gate_t.T.reshape(R, 1)
    routed = _make_gather_kernel()(x2d, fidx)
    out = pl.pallas_call(
        _expert_kernel,
        grid=(E,),
        in_specs=[
            pl.BlockSpec(memory_space=pltpu.SMEM),
            pl.BlockSpec((B * M, D), lambda e: (e, 0)),
            pl.BlockSpec((1, D, D), lambda e: (e, 0, 0)),
            pl.BlockSpec((B * M, 1), lambda e: (e, 0)),
        ],
        out_specs=pl.BlockSpec((B * N, D), lambda e: (0, 0)),
        out_shape=jax.ShapeDtypeStruct((B * N, D), jnp.float32),
        scratch_shapes=[pltpu.VMEM((B * M, D), jnp.float32)],
        compiler_params=pltpu.CompilerParams(
            dimension_semantics=("arbitrary",)),
    )(fidx, routed, experts, gates)
    return out.reshape(B, N, D)
